# baseline (speedup 1.0000x reference)
import functools

import jax
import jax.numpy as jnp
from jax import lax
from jax.experimental import pallas as pl
from jax.experimental.pallas import tpu as pltpu
from jax.experimental.pallas import tpu_sc as plsc

N = 10000
D = 128
E = 320000

NUM_SUBCORES = 16

CHUNK = 128
N_PAD = 10240
E_PAD = 327680
N_CHUNKS = E_PAD // CHUNK
CPW = N_CHUNKS // NUM_SUBCORES
ROWS_PER_TILE = N_PAD // NUM_SUBCORES
LANES = 16
NSLOT = 4
NSTEP = CPW // NSLOT


def _sc_aggregate(table, src2d, dst2d, with_deg):
    mesh = plsc.VectorSubcoreMesh(core_axis_name="c", subcore_axis_name="s", num_cores=1)

    out_type = [jax.ShapeDtypeStruct((N_PAD, D), jnp.float32)]
    scratch = (
        [pltpu.VMEM((CHUNK,), jnp.int32) for _ in range(8)]
        + [pltpu.VMEM((CHUNK, D), jnp.float32) for _ in range(2)]
        + [pltpu.VMEM_SHARED((N_PAD, D), jnp.float32)]
        + [pltpu.SemaphoreType.DMA for _ in range(12)]
    )
    if with_deg:
        out_type.append(jax.ShapeDtypeStruct((N_PAD,), jnp.float32))
        out_type.append(jax.ShapeDtypeStruct(
            (NUM_SUBCORES, N_PAD), jnp.float32))
        scratch += [
            pltpu.VMEM((N_PAD,), jnp.float32),
            pltpu.VMEM((ROWS_PER_TILE,), jnp.float32),
            pltpu.VMEM((ROWS_PER_TILE,), jnp.float32),
        ]

    @functools.partial(
        pl.kernel, mesh=mesh,
        compiler_params=pltpu.CompilerParams(needs_layout_passes=False),
        out_type=out_type, scratch_types=scratch)
    def k(table_hbm, src_hbm, dst_hbm, agg_out, *rest):
        if with_deg:
            deg_out, stage = rest[0], rest[1]
            rest = rest[2:]
            deg_v, acc_v, tmp_v = rest[23:]
        isl = rest[0:4]
        dsl = rest[4:8]
        rws = rest[8:10]
        agg_sh = rest[10]
        isem = rest[11:15]
        dsem = rest[15:19]
        gsem = rest[19:21]
        ssem = rest[21:23]

        cid = lax.axis_index("c")
        sid = lax.axis_index("s")

        @pl.when(cid == 0)
        def _body():
            row0 = sid * ROWS_PER_TILE

            def zrow(i, c):
                def zcol(j, c2):
                    rws[0][i, pl.ds(j * LANES, LANES)] = jnp.zeros(
                        (LANES,), jnp.float32)
                    return c2
                lax.fori_loop(0, D // LANES, zcol, 0)
                return c
            lax.fori_loop(0, CHUNK, zrow, 0)

            def zstripe(s, c):
                pltpu.sync_copy(rws[0],
                                agg_sh.at[pl.ds(row0 + s * CHUNK, CHUNK)])
                return c
            lax.fori_loop(0, ROWS_PER_TILE // CHUNK, zstripe, 0)
            if with_deg:
                def zb(j, c):
                    deg_v[pl.ds(j * LANES, LANES)] = jnp.zeros((LANES,),
                                                               jnp.float32)
                    return c
                lax.fori_loop(0, N_PAD // LANES, zb, 0)
            plsc.subcore_barrier()

            chunk0 = sid * CPW

            def hist(idx_ref):
                def hb(j, c2):
                    iv = idx_ref[pl.ds(j * LANES, LANES)]
                    plsc.addupdate_scatter(
                        deg_v, [iv], jnp.ones((LANES,), jnp.float32))
                    return c2
                lax.fori_loop(0, CHUNK // LANES, hb, 0)

            for s in range(NSLOT):
                pltpu.async_copy(src_hbm.at[chunk0 + s], isl[s], isem[s])
                pltpu.async_copy(dst_hbm.at[chunk0 + s], dsl[s], dsem[s])

            def step(g, carry):
                for b in range(NSLOT):
                    r = b % 2
                    c = chunk0 + g * NSLOT + b

                    def drain():
                        pltpu.make_async_copy(
                            table_hbm.at[pl.ds(0, CHUNK)], rws[r],
                            ssem[r]).wait()

                    def refill():
                        s2 = (b + 2) % NSLOT
                        pltpu.async_copy(src_hbm.at[c + 2], isl[s2],
                                         isem[s2])
                        pltpu.async_copy(dst_hbm.at[c + 2], dsl[s2],
                                         dsem[s2])

                    if b < 2:
                        @pl.when(g > 0)
                        def _():
                            drain()
                            refill()
                    else:
                        drain()
                        @pl.when(g < NSTEP - 1)
                        def _():
                            refill()

                    pltpu.make_async_copy(src_hbm.at[c], isl[b],
                                          isem[b]).wait()
                    pltpu.make_async_copy(dst_hbm.at[c], dsl[b],
                                          dsem[b]).wait()
                    pltpu.async_copy(table_hbm.at[isl[b]], rws[r], gsem[r])

                    rp = (b + 1) % 2
                    bp = (b - 1) % NSLOT

                    def consume_prev():
                        pltpu.make_async_copy(
                            table_hbm.at[pl.ds(0, CHUNK)], rws[rp],
                            gsem[rp]).wait()
                        if with_deg:
                            hist(dsl[bp])
                        pltpu.async_copy(rws[rp], agg_sh.at[dsl[bp]],
                                         ssem[rp], add=True)
                    if b == 0:
                        @pl.when(g > 0)
                        def _():
                            consume_prev()
                    else:
                        consume_prev()
                return carry

            lax.fori_loop(0, NSTEP, step, 0)

            pltpu.make_async_copy(table_hbm.at[pl.ds(0, CHUNK)], rws[1],
                                  gsem[1]).wait()
            if with_deg:
                hist(dsl[3])
            pltpu.async_copy(rws[1], agg_sh.at[dsl[3]], ssem[1], add=True)
            for r in range(2):
                pltpu.make_async_copy(table_hbm.at[pl.ds(0, CHUNK)],
                                      rws[r], ssem[r]).wait()

            if with_deg:
                pltpu.sync_copy(deg_v, stage.at[sid])
            plsc.subcore_barrier()

            pltpu.sync_copy(agg_sh.at[pl.ds(row0, ROWS_PER_TILE)],
                            agg_out.at[pl.ds(row0, ROWS_PER_TILE)])

            if with_deg:
                def zb2(j, c):
                    acc_v[pl.ds(j * LANES, LANES)] = jnp.zeros(
                        (LANES,), jnp.float32)
                    return c
                lax.fori_loop(0, ROWS_PER_TILE // LANES, zb2, 0)

                def rb(t, c):
                    pltpu.sync_copy(
                        stage.at[t, pl.ds(row0, ROWS_PER_TILE)], tmp_v)

                    def ab(j, c2):
                        s = pl.ds(j * LANES, LANES)
                        acc_v[s] = acc_v[s] + tmp_v[s]
                        return c2
                    lax.fori_loop(0, ROWS_PER_TILE // LANES, ab, 0)
                    return c
                lax.fori_loop(0, NUM_SUBCORES, rb, 0)
                pltpu.sync_copy(acc_v,
                                deg_out.at[pl.ds(row0, ROWS_PER_TILE)])

    res = k(table, src2d, dst2d)
    if not isinstance(res, (list, tuple)):
        res = (res,)
    return res


BLK = 1024
GRID = N_PAD // BLK


def _tc_layer1(agg, deg, W, b2d):
    def body(a0, d0, w, b, h_ref, dinv_ref):
        dinv = 1.0 / jnp.maximum(d0[...], 1.0)
        a = a0[...] * dinv
        h = jnp.dot(a, w[...], preferred_element_type=jnp.float32) + b[...]
        h_ref[...] = jnp.maximum(h, 0.0)
        dinv_ref[...] = dinv

    row_spec = pl.BlockSpec((BLK, D), lambda i: (i, 0))
    col_spec = pl.BlockSpec((BLK, 1), lambda i: (i, 0))
    return pl.pallas_call(
        body,
        grid=(GRID,),
        in_specs=[row_spec, col_spec,
                  pl.BlockSpec((D, D), lambda i: (0, 0)),
                  pl.BlockSpec((1, D), lambda i: (0, 0))],
        out_specs=[row_spec, col_spec],
        out_shape=[jax.ShapeDtypeStruct((N_PAD, D), jnp.float32),
                   jax.ShapeDtypeStruct((N_PAD, 1), jnp.float32)],
    )(agg, deg, W, b2d)


def _tc_layer2(agg, dinv, W, b2d):
    def body(a0, dv, w, b, o_ref):
        a = a0[...] * dv[...]
        o_ref[...] = jnp.dot(a, w[...],
                             preferred_element_type=jnp.float32) + b[...]

    row_spec = pl.BlockSpec((BLK, D), lambda i: (i, 0))
    col_spec = pl.BlockSpec((BLK, 1), lambda i: (i, 0))
    return pl.pallas_call(
        body,
        grid=(GRID,),
        in_specs=[row_spec, col_spec,
                  pl.BlockSpec((D, D), lambda i: (0, 0)),
                  pl.BlockSpec((1, D), lambda i: (0, 0))],
        out_specs=row_spec,
        out_shape=jax.ShapeDtypeStruct((N_PAD, D), jnp.float32),
    )(agg, dinv, W, b2d)


def kernel(x, edge_index, W1, b1, W2, b2):
    src = edge_index[0]
    dst = edge_index[1]

    pad = E_PAD - E
    src_pad = jnp.concatenate(
        [src, jnp.zeros((pad,), jnp.int32)]).reshape(N_CHUNKS, CHUNK)
    dst_pad = jnp.concatenate(
        [dst, jnp.full((pad,), N, jnp.int32)]).reshape(N_CHUNKS, CHUNK)

    agg1, deg, _ = _sc_aggregate(x, src_pad, dst_pad, True)

    h, dinv = _tc_layer1(agg1, deg.reshape(N_PAD, 1), W1, b1.reshape(1, D))

    (agg2,) = _sc_aggregate(h, src_pad, dst_pad, False)

    out = _tc_layer2(agg2, dinv, W2, b2.reshape(1, D))
    return out[:N]


# speedup vs baseline: 1.2592x; 1.2592x over previous
import functools

import jax
import jax.numpy as jnp
from jax import lax
from jax.experimental import pallas as pl
from jax.experimental.pallas import tpu as pltpu
from jax.experimental.pallas import tpu_sc as plsc

N = 10000
D = 128
E = 320000

NUM_CORES = 2
NUM_SUBCORES = 16

CHUNK = 128
GRP = 8
N_PAD = 10240
E_PAD = 327680
N_CHUNKS = E_PAD // CHUNK
N_GROUPS = N_CHUNKS // GRP
ROWS_PER_TILE = N_PAD // NUM_SUBCORES
LANES = 16

G0 = 288
G1 = N_GROUPS - G0
GPW0 = G0 // NUM_SUBCORES
GPW1 = G1 // NUM_SUBCORES


def _sc_aggregate(table, src3d, dst3d, with_deg):
    mesh = plsc.VectorSubcoreMesh(core_axis_name="c", subcore_axis_name="s")

    out_type = [jax.ShapeDtypeStruct((NUM_CORES, N_PAD, D), jnp.float32)]
    scratch = (
        [pltpu.VMEM((GRP, CHUNK), jnp.int32) for _ in range(4)]
        + [pltpu.VMEM((CHUNK, D), jnp.float32) for _ in range(2)]
        + [pltpu.VMEM_SHARED((N_PAD, D), jnp.float32)]
        + [pltpu.SemaphoreType.DMA for _ in range(8)]
    )
    if with_deg:
        out_type.append(jax.ShapeDtypeStruct((NUM_CORES, N_PAD), jnp.float32))
        out_type.append(jax.ShapeDtypeStruct(
            (NUM_CORES, NUM_SUBCORES, N_PAD), jnp.float32))
        scratch += [
            pltpu.VMEM((N_PAD,), jnp.float32),
            pltpu.VMEM((ROWS_PER_TILE,), jnp.float32),
            pltpu.VMEM((ROWS_PER_TILE,), jnp.float32),
        ]

    @functools.partial(
        pl.kernel, mesh=mesh,
        compiler_params=pltpu.CompilerParams(needs_layout_passes=False),
        out_type=out_type, scratch_types=scratch)
    def k(table_hbm, src_hbm, dst_hbm, agg_out, *rest):
        if with_deg:
            deg_out, stage = rest[0], rest[1]
            rest = rest[2:]
            deg_v, acc_v, tmp_v = rest[15:]
        isg = rest[0:2]
        dsg = rest[2:4]
        rws = rest[4:6]
        agg_sh = rest[6]
        isem = rest[7:9]
        dsem = rest[9:11]
        gsem = rest[11:13]
        ssem = rest[13:15]

        cid = lax.axis_index("c")
        sid = lax.axis_index("s")
        row0 = sid * ROWS_PER_TILE

        def zrow(i, c):
            def zcol(j, c2):
                rws[0][i, pl.ds(j * LANES, LANES)] = jnp.zeros((LANES,),
                                                               jnp.float32)
                return c2
            lax.fori_loop(0, D // LANES, zcol, 0)
            return c
        lax.fori_loop(0, CHUNK, zrow, 0)

        def zstripe(s, c):
            pltpu.sync_copy(rws[0],
                            agg_sh.at[pl.ds(row0 + s * CHUNK, CHUNK)])
            return c
        lax.fori_loop(0, ROWS_PER_TILE // CHUNK, zstripe, 0)
        if with_deg:
            def zb(j, c):
                deg_v[pl.ds(j * LANES, LANES)] = jnp.zeros((LANES,),
                                                           jnp.float32)
                return c
            lax.fori_loop(0, N_PAD // LANES, zb, 0)
        plsc.subcore_barrier()

        gbase = jnp.where(cid == 0, sid * GPW0, G0 + sid * GPW1)
        ngrp = jnp.where(cid == 0, GPW0, GPW1)
        usup = ngrp // 2

        def hist(idx_ref):
            def hb(j, c2):
                iv = idx_ref[pl.ds(j * LANES, LANES)]
                plsc.addupdate_scatter(
                    deg_v, [iv], jnp.ones((LANES,), jnp.float32))
                return c2
            lax.fori_loop(0, CHUNK // LANES, hb, 0)

        pltpu.async_copy(src_hbm.at[gbase], isg[0], isem[0])
        pltpu.async_copy(dst_hbm.at[gbase], dsg[0], dsem[0])

        def idx_wait(p):
            pltpu.make_async_copy(src_hbm.at[gbase], isg[p],
                                  isem[p]).wait()
            pltpu.make_async_copy(dst_hbm.at[gbase], dsg[p],
                                  dsem[p]).wait()

        def drain_scatter(r):
            pltpu.make_async_copy(table_hbm.at[pl.ds(0, CHUNK)], rws[r],
                                  ssem[r]).wait()

        def consume(rp, d_ref):
            pltpu.make_async_copy(table_hbm.at[pl.ds(0, CHUNK)], rws[rp],
                                  gsem[rp]).wait()
            if with_deg:
                hist(d_ref)
            pltpu.async_copy(rws[rp], agg_sh.at[d_ref], ssem[rp], add=True)

        def step(u, carry):
            for p in range(2):
                gi = gbase + 2 * u + p
                for j in range(GRP):
                    r = j % 2
                    rp = (j + 1) % 2
                    first = (p == 0 and j == 0)
                    second = (p == 0 and j == 1)

                    if first:
                        idx_wait(0)

                        @pl.when(u > 0)
                        def _():
                            drain_scatter(0)
                    elif p == 1 and j == 0:
                        idx_wait(1)
                        drain_scatter(0)
                    elif second:
                        @pl.when(u > 0)
                        def _():
                            drain_scatter(1)
                    else:
                        drain_scatter(r)

                    pltpu.async_copy(table_hbm.at[isg[p].at[j]], rws[r],
                                     gsem[r])

                    if first:
                        @pl.when(u > 0)
                        def _():
                            consume(1, dsg[1].at[GRP - 1])
                    elif j == 0:
                        consume(1, dsg[p ^ 1].at[GRP - 1])
                    else:
                        consume(rp, dsg[p].at[j - 1])

                    if j == 2:
                        @pl.when(2 * u + p + 1 < ngrp)
                        def _():
                            pltpu.async_copy(src_hbm.at[gi + 1],
                                             isg[p ^ 1], isem[p ^ 1])
                            pltpu.async_copy(dst_hbm.at[gi + 1],
                                             dsg[p ^ 1], dsem[p ^ 1])
            return carry

        lax.fori_loop(0, usup, step, 0)

        consume(1, dsg[1].at[GRP - 1])
        for r in range(2):
            drain_scatter(r)

        if with_deg:
            pltpu.sync_copy(deg_v, stage.at[cid, sid])
        plsc.subcore_barrier()

        pltpu.sync_copy(agg_sh.at[pl.ds(row0, ROWS_PER_TILE)],
                        agg_out.at[cid, pl.ds(row0, ROWS_PER_TILE)])

        if with_deg:
            def zb2(j, c):
                acc_v[pl.ds(j * LANES, LANES)] = jnp.zeros((LANES,),
                                                           jnp.float32)
                return c
            lax.fori_loop(0, ROWS_PER_TILE // LANES, zb2, 0)

            def rb(t, c):
                pltpu.sync_copy(stage.at[cid, t, pl.ds(row0,
                                                       ROWS_PER_TILE)],
                                tmp_v)

                def ab(j, c2):
                    s = pl.ds(j * LANES, LANES)
                    acc_v[s] = acc_v[s] + tmp_v[s]
                    return c2
                lax.fori_loop(0, ROWS_PER_TILE // LANES, ab, 0)
                return c
            lax.fori_loop(0, NUM_SUBCORES, rb, 0)
            pltpu.sync_copy(acc_v, deg_out.at[cid, pl.ds(row0,
                                                         ROWS_PER_TILE)])

    res = k(table, src3d, dst3d)
    if not isinstance(res, (list, tuple)):
        res = (res,)
    return res


BLK = 1024
GRID = N_PAD // BLK


def _tc_layer1(agg0, agg1, deg0, deg1, W, b2d):
    def body(a0, a1, d0, d1, w, b, h_ref, dinv_ref):
        deg = d0[...] + d1[...]
        dinv = 1.0 / jnp.maximum(deg, 1.0)
        a = (a0[...] + a1[...]) * dinv
        h = jnp.dot(a, w[...], preferred_element_type=jnp.float32) + b[...]
        h_ref[...] = jnp.maximum(h, 0.0)
        dinv_ref[...] = dinv

    row_spec = pl.BlockSpec((BLK, D), lambda i: (i, 0))
    col_spec = pl.BlockSpec((BLK, 1), lambda i: (i, 0))
    return pl.pallas_call(
        body,
        grid=(GRID,),
        in_specs=[row_spec, row_spec, col_spec, col_spec,
                  pl.BlockSpec((D, D), lambda i: (0, 0)),
                  pl.BlockSpec((1, D), lambda i: (0, 0))],
        out_specs=[row_spec, col_spec],
        out_shape=[jax.ShapeDtypeStruct((N_PAD, D), jnp.float32),
                   jax.ShapeDtypeStruct((N_PAD, 1), jnp.float32)],
    )(agg0, agg1, deg0, deg1, W, b2d)


def _tc_layer2(agg0, agg1, dinv, W, b2d):
    def body(a0, a1, dv, w, b, o_ref):
        a = (a0[...] + a1[...]) * dv[...]
        o_ref[...] = jnp.dot(a, w[...],
                             preferred_element_type=jnp.float32) + b[...]

    row_spec = pl.BlockSpec((BLK, D), lambda i: (i, 0))
    col_spec = pl.BlockSpec((BLK, 1), lambda i: (i, 0))
    return pl.pallas_call(
        body,
        grid=(GRID,),
        in_specs=[row_spec, row_spec, col_spec,
                  pl.BlockSpec((D, D), lambda i: (0, 0)),
                  pl.BlockSpec((1, D), lambda i: (0, 0))],
        out_specs=row_spec,
        out_shape=jax.ShapeDtypeStruct((N_PAD, D), jnp.float32),
    )(agg0, agg1, dinv, W, b2d)


def kernel(x, edge_index, W1, b1, W2, b2):
    src = edge_index[0]
    dst = edge_index[1]

    pad = E_PAD - E
    src_pad = jnp.concatenate(
        [src, jnp.zeros((pad,), jnp.int32)]).reshape(N_GROUPS, GRP, CHUNK)
    dst_pad = jnp.concatenate(
        [dst, jnp.full((pad,), N, jnp.int32)]).reshape(N_GROUPS, GRP, CHUNK)

    agg1p, degp, _ = _sc_aggregate(x, src_pad, dst_pad, True)

    h, dinv = _tc_layer1(agg1p[0], agg1p[1],
                         degp[0].reshape(N_PAD, 1), degp[1].reshape(N_PAD, 1),
                         W1, b1.reshape(1, D))

    (agg2p,) = _sc_aggregate(h, src_pad, dst_pad, False)

    out = _tc_layer2(agg2p[0], agg2p[1], dinv, W2, b2.reshape(1, D))
    return out[:N]
